# baseline (speedup 1.0000x reference)
import functools

import jax
import jax.numpy as jnp
from jax import lax
from jax.experimental import pallas as pl
from jax.experimental.pallas import tpu as pltpu
from jax.experimental.pallas import tpu_sc as plsc

N_BLOCKS = 3
HEADS = 4
OUT = 32
HID = 128
HALF = HID // 2
NEG_SLOPE = 0.2

NC = 2
NS = 16
LANES = 16
CHUNK = 80
GBLK = 50
NGBLK = 5


def _f32(x):
    return x.astype(jnp.float32)


_GD = lax.GatherDimensionNumbers(offset_dims=(), collapsed_slice_dims=(0,),
                                 start_index_map=(0,))


def _lane_splat(v, idx16):
    return lax.gather(v, idx16[:, None], _GD, slice_sizes=(1,),
                      mode=lax.GatherScatterMode.PROMISE_IN_BOUNDS)


def _tc_a_body(x_ref, wg_ref, asel_ref, adsel_ref, rsel_ref, psel_ref,
               h2_ref, alpha_ref, hinit_ref, dinit_ref):
    x = x_ref[...]
    h0 = jnp.dot(x, wg_ref[...], preferred_element_type=jnp.float32)
    h2_ref[0] = h0[:, :HALF]
    h2_ref[1] = h0[:, HALF:]
    alpha_s = jnp.dot(h0, asel_ref[...], preferred_element_type=jnp.float32)
    alpha_d = jnp.dot(h0, adsel_ref[...], preferred_element_type=jnp.float32)
    alpha_ref[0] = jnp.concatenate([alpha_s[:, 0:2], alpha_d[:, 0:2]], axis=1)
    alpha_ref[1] = jnp.concatenate([alpha_s[:, 2:4], alpha_d[:, 2:4]], axis=1)
    e = alpha_s + alpha_d
    e = jnp.where(e > 0, e, NEG_SLOPE * e)
    w_self = jnp.exp(e)
    w_rep = jnp.dot(w_self, rsel_ref[...], preferred_element_type=jnp.float32)
    hw = h0 * w_rep
    hinit_ref[0] = hw[:, :HALF]
    hinit_ref[1] = hw[:, HALF:]
    hid4 = lax.broadcasted_iota(jnp.int32, w_self.shape, 1)
    d0 = jnp.dot(jnp.where(hid4 < 2, w_self, 0.0), psel_ref[...],
                 preferred_element_type=jnp.float32)
    d1 = jnp.dot(jnp.where(hid4 >= 2, w_self, 0.0), psel_ref[...],
                 preferred_element_type=jnp.float32)
    dinit_ref[0] = d0
    dinit_ref[1] = d1


def _tc_b_body(acch_ref, accd_ref, wm_ref, psum_ref, h_ref, v_ref):
    num = jnp.concatenate([acch_ref[0], acch_ref[1]], axis=1)
    den = accd_ref[0] + accd_ref[1]
    den_rep = jnp.dot(den, psum_ref[...], preferred_element_type=jnp.float32)
    h = num / (den_rep + 1e-16)
    h_ref[...] = h
    v = jnp.dot(h, wm_ref[...], preferred_element_type=jnp.float32)
    v_ref[0] = v[:, :HALF]
    v_ref[1] = v[:, HALF:]


def _gru_math(m, h, wi, wh, bi, bh):
    gi = jnp.dot(m, wi, preferred_element_type=jnp.float32) + bi
    gh = jnp.dot(h, wh, preferred_element_type=jnp.float32) + bh
    ir, iz, ig = gi[:, :HID], gi[:, HID:2 * HID], gi[:, 2 * HID:]
    hr, hz, hg = gh[:, :HID], gh[:, HID:2 * HID], gh[:, 2 * HID:]
    r = jax.nn.sigmoid(ir + hr)
    z = jax.nn.sigmoid(iz + hz)
    g = jnp.tanh(ig + r * hg)
    return (1.0 - z) * g + z * h


def _tc_gru_body(accm_ref, h_ref, hist_ref, wi_ref, wh_ref, bi_ref, bh_ref,
                 wmn_ref, hn_ref, histn_ref, vn_ref):
    m = jnp.concatenate([accm_ref[0], accm_ref[1]], axis=1)
    hn = _gru_math(m, h_ref[...], wi_ref[...], wh_ref[...], bi_ref[...],
                   bh_ref[...])
    hn_ref[...] = hn
    histn_ref[...] = hist_ref[...] + hn
    v = jnp.dot(hn, wmn_ref[...], preferred_element_type=jnp.float32)
    vn_ref[0] = v[:, :HALF]
    vn_ref[1] = v[:, HALF:]


def _tc_gru_last_body(accm_ref, h_ref, hist_ref, wi_ref, wh_ref, bi_ref,
                      bh_ref, histn_ref):
    m = jnp.concatenate([accm_ref[0], accm_ref[1]], axis=1)
    hn = _gru_math(m, h_ref[...], wi_ref[...], wh_ref[...], bi_ref[...],
                   bh_ref[...])
    histn_ref[...] = hist_ref[...] + hn


def _tc_final_body(hist_ref, batch_ref, y_ref, w1_ref, b1_ref, w2_ref, b2_ref,
                   scores_ref, loss_ref):
    n = hist_ref.shape[0]
    g = y_ref.shape[0]
    batch = batch_ref[...]
    gids = lax.broadcasted_iota(jnp.int32, (g, n), 0)
    sel = jnp.where(gids == batch, 1.0, 0.0).astype(jnp.float32)
    pooled = jnp.dot(sel, hist_ref[...], preferred_element_type=jnp.float32)
    z = jnp.maximum(
        jnp.dot(pooled, w1_ref[...], preferred_element_type=jnp.float32)
        + b1_ref[...], 0.0)
    s = jnp.dot(z, w2_ref[...], preferred_element_type=jnp.float32) \
        + b2_ref[...]
    scores_ref[...] = s
    y = y_ref[...]
    lv = jnp.maximum(s, 0.0) - s * y + jnp.log1p(jnp.exp(-jnp.abs(s)))
    loss_ref[...] = jnp.sum(lv).reshape(1, 1) / g


def _tc_call(body, out_shapes, *args):
    return pl.pallas_call(
        body, out_shape=out_shapes,
        compiler_params=pltpu.CompilerParams(
            vmem_limit_bytes=100 * 1024 * 1024))(*args)


def _segsum_sc(v2, src3d, dst3d, n):
    chunks_tile = src3d.shape[1]
    rows_tile = n // NS
    mesh = plsc.VectorSubcoreMesh(core_axis_name="c", subcore_axis_name="s",
                                  num_cores=NC, num_subcores=NS)

    @functools.partial(
        pl.kernel,
        out_type=jax.ShapeDtypeStruct((NC, n, HALF), jnp.float32),
        mesh=mesh,
        compiler_params=pltpu.CompilerParams(needs_layout_passes=False,
                                             use_tc_tiling_on_sc=False),
        scratch_types=[
            pltpu.VMEM((chunks_tile, CHUNK), jnp.int32),
            pltpu.VMEM((chunks_tile, CHUNK), jnp.int32),
            pltpu.VMEM((CHUNK, HALF), jnp.float32),
            pltpu.VMEM((CHUNK, HALF), jnp.float32),
            pltpu.VMEM_SHARED((n, HALF), jnp.float32),
            pltpu.SemaphoreType.DMA,
            pltpu.SemaphoreType.DMA,
        ],
    )
    def k(v_hbm, src_hbm, dst_hbm, z_hbm, out_hbm,
          sidx, didx, rows_a, rows_b, acc, sem_a, sem_b):
        cid = lax.axis_index("c")
        sid = lax.axis_index("s")
        pltpu.sync_copy(src_hbm.at[sid], sidx)
        pltpu.sync_copy(dst_hbm.at[sid], didx)
        rbase = sid * rows_tile
        pltpu.sync_copy(z_hbm.at[pl.ds(rbase, rows_tile)],
                        acc.at[pl.ds(rbase, rows_tile)])
        plsc.subcore_barrier()

        def gather(c, rows, sem):
            return pltpu.async_copy(v_hbm.at[cid].at[sidx.at[c]], rows, sem)

        def scat(c, rows):
            pltpu.sync_copy(rows, acc.at[didx.at[c]], add=True)

        last = chunks_tile - 1
        gather(0, rows_a, sem_a).wait()

        def body(i, _):
            c = 2 * i
            db = gather(c + 1, rows_b, sem_b)
            scat(c, rows_a)
            da = gather(jnp.minimum(c + 2, last), rows_a, sem_a)
            db.wait()
            scat(c + 1, rows_b)
            da.wait()
            return 0

        lax.fori_loop(0, chunks_tile // 2, body, 0)
        plsc.subcore_barrier()
        pltpu.sync_copy(acc.at[pl.ds(rbase, rows_tile)],
                        out_hbm.at[cid, pl.ds(rbase, rows_tile)])

    zeros = jnp.zeros((n, HALF), jnp.float32)
    return k(v2, src3d, dst3d, zeros)


def _gat_edges_sc(h2, alpha2, src4d, dst4d, hinit2, dinit2, n):
    nblk = src4d.shape[1]
    gblk = src4d.shape[2]
    rows_tile = n // NS
    mesh = plsc.VectorSubcoreMesh(core_axis_name="c", subcore_axis_name="s",
                                  num_cores=NC, num_subcores=NS)
    groups = CHUNK // LANES

    @functools.partial(
        pl.kernel,
        out_type=(jax.ShapeDtypeStruct((NC, n, HALF), jnp.float32),
                  jax.ShapeDtypeStruct((NC, n, LANES), jnp.float32)),
        mesh=mesh,
        compiler_params=pltpu.CompilerParams(needs_layout_passes=False,
                                             use_tc_tiling_on_sc=False),
        scratch_types=[
            pltpu.VMEM((n * 4,), jnp.float32),
            pltpu.VMEM((GBLK, CHUNK), jnp.int32),
            pltpu.VMEM((GBLK, CHUNK), jnp.int32),
            pltpu.VMEM((CHUNK, HALF), jnp.float32),
            pltpu.VMEM((CHUNK, HALF), jnp.float32),
            pltpu.VMEM((CHUNK, HALF), jnp.float32),
            pltpu.VMEM((CHUNK, LANES), jnp.float32),
            pltpu.VMEM_SHARED((n, HALF), jnp.float32),
            pltpu.VMEM_SHARED((n, LANES), jnp.float32),
            pltpu.SemaphoreType.DMA,
            pltpu.SemaphoreType.DMA,
        ],
    )
    def k(h_hbm, alpha_hbm, src_hbm, dst_hbm, hinit_hbm, dinit_hbm,
          outh_hbm, outd_hbm,
          alpha_v, sidx, didx, rows_a, rows_b, rows_s, wrow, acch, accd,
          sem_a, sem_b):
        cid = lax.axis_index("c")
        sid = lax.axis_index("s")
        pltpu.sync_copy(alpha_hbm.at[cid], alpha_v)
        rbase = sid * rows_tile
        pltpu.sync_copy(hinit_hbm.at[cid, pl.ds(rbase, rows_tile)],
                        acch.at[pl.ds(rbase, rows_tile)])
        pltpu.sync_copy(dinit_hbm.at[cid, pl.ds(rbase, rows_tile)],
                        accd.at[pl.ds(rbase, rows_tile)])
        zf = jnp.zeros((LANES,), jnp.float32)
        for kk in range(CHUNK):
            wrow[kk, :] = zf
        plsc.subcore_barrier()

        def gather(c, rows, sem):
            return pltpu.async_copy(h_hbm.at[cid].at[sidx.at[c]], rows, sem)

        def scale(c, rows):
            lane = lax.iota(jnp.int32, LANES)
            for g in range(groups):
                el = g * LANES + lane
                s_i = sidx[c, pl.ds(g * LANES, LANES)]
                d_i = didx[c, pl.ds(g * LANES, LANES)]
                ws = []
                for hl in range(2):
                    a_s = plsc.load_gather(alpha_v, [s_i * 4 + hl])
                    a_d = plsc.load_gather(alpha_v, [d_i * 4 + 2 + hl])
                    e = a_s + a_d
                    e = jnp.where(e > 0, e, NEG_SLOPE * e)
                    w = jnp.exp(e)
                    ws.append(w)
                    wcol = jnp.full((LANES,), hl, jnp.int32) + 2 * cid
                    plsc.store_scatter(wrow, [el, wcol], w)
                for k in range(LANES):
                    ek = g * LANES + k
                    wv = wrow[ek, pl.ds(0, LANES)]
                    for hl in range(2):
                        wk = _lane_splat(
                            wv, jnp.full((LANES,), hl, jnp.int32) + 2 * cid)
                        base = hl * OUT
                        rows_s[ek, pl.ds(base, LANES)] = (
                            rows[ek, pl.ds(base, LANES)] * wk)
                        rows_s[ek, pl.ds(base + LANES, LANES)] = (
                            rows[ek, pl.ds(base + LANES, LANES)] * wk)

        def scat(c, rows):
            pltpu.sync_copy(rows_s, acch.at[didx.at[c]], add=True)
            pltpu.sync_copy(wrow, accd.at[didx.at[c]], add=True)

        def blk(b, _):
            pltpu.sync_copy(src_hbm.at[sid, b], sidx)
            pltpu.sync_copy(dst_hbm.at[sid, b], didx)
            gather(0, rows_a, sem_a).wait()

            def body(i, _):
                c = 2 * i
                db = gather(c + 1, rows_b, sem_b)
                scale(c, rows_a)
                scat(c, rows_a)
                da = gather(jnp.minimum(c + 2, gblk - 1), rows_a, sem_a)
                db.wait()
                scale(c + 1, rows_b)
                scat(c + 1, rows_b)
                da.wait()
                return 0

            lax.fori_loop(0, gblk // 2, body, 0)
            return 0

        lax.fori_loop(0, nblk, blk, 0)
        plsc.subcore_barrier()
        pltpu.sync_copy(acch.at[pl.ds(rbase, rows_tile)],
                        outh_hbm.at[cid, pl.ds(rbase, rows_tile)])
        pltpu.sync_copy(accd.at[pl.ds(rbase, rows_tile)],
                        outd_hbm.at[cid, pl.ds(rbase, rows_tile)])

    return k(h2, alpha2, src4d, dst4d, hinit2, dinit2)


def kernel(x, edge_index, batch, y, num_graphs, W_gat, a_src, a_dst, Wm, Wi,
           Wh, bi, bh, W1, b1, W2, b2):
    n = x.shape[0]
    e_cnt = edge_index.shape[1]
    g_cnt = y.shape[0]
    del num_graphs

    eye4 = jnp.eye(HEADS, dtype=jnp.float32)
    asel = jnp.einsum('hk,hg->hkg', _f32(a_src), eye4).reshape(HID, HEADS)
    adsel = jnp.einsum('hk,hg->hkg', _f32(a_dst), eye4).reshape(HID, HEADS)
    rsel = jnp.repeat(eye4, OUT, axis=1).reshape(HEADS, HID)
    psel = jnp.concatenate(
        [eye4, jnp.zeros((HEADS, LANES - HEADS), jnp.float32)], axis=1)
    psum = jnp.concatenate(
        [jnp.repeat(eye4, OUT, axis=1).reshape(HEADS, HID),
         jnp.zeros((LANES - HEADS, HID), jnp.float32)], axis=0)

    n_pad = ((n + 127) // 128) * 128
    chunks_tile = e_cnt // (NS * CHUNK)
    src = edge_index[0].astype(jnp.int32)
    dst = edge_index[1].astype(jnp.int32)
    src3d = src.reshape(NS, chunks_tile, CHUNK)
    dst3d = dst.reshape(NS, chunks_tile, CHUNK)
    src4d = src.reshape(NS, NGBLK, GBLK, CHUNK)
    dst4d = dst.reshape(NS, NGBLK, GBLK, CHUNK)
    x_p = jnp.pad(_f32(x), ((0, n_pad - n), (0, 0)))
    batch2d = jnp.pad(batch.astype(jnp.int32), (0, n_pad - n),
                      constant_values=g_cnt).reshape(1, n_pad)
    y2d = _f32(y).reshape(g_cnt, 1)
    bi2 = _f32(bi).reshape(N_BLOCKS, 1, 3 * HID)
    bh2 = _f32(bh).reshape(N_BLOCKS, 1, 3 * HID)
    b12 = _f32(b1).reshape(1, 64)
    b22 = _f32(b2).reshape(1, 1)

    h2, alpha2, hinit2, dinit2 = _tc_call(
        _tc_a_body,
        (jax.ShapeDtypeStruct((NC, n_pad, HALF), jnp.float32),
         jax.ShapeDtypeStruct((NC, n_pad, HEADS), jnp.float32),
         jax.ShapeDtypeStruct((NC, n_pad, HALF), jnp.float32),
         jax.ShapeDtypeStruct((NC, n_pad, LANES), jnp.float32)),
        x_p, _f32(W_gat), asel, adsel, rsel, psel)

    acch, accd = _gat_edges_sc(h2, alpha2.reshape(NC, n_pad * HEADS), src4d,
                               dst4d, hinit2, dinit2, n_pad)

    h, v2 = _tc_call(
        _tc_b_body,
        (jax.ShapeDtypeStruct((n_pad, HID), jnp.float32),
         jax.ShapeDtypeStruct((NC, n_pad, HALF), jnp.float32)),
        acch, accd, _f32(Wm[0]), psum)
    hist = h

    for i in range(N_BLOCKS):
        accm = _segsum_sc(v2, src3d, dst3d, n_pad)
        if i + 1 < N_BLOCKS:
            h, hist, v2 = _tc_call(
                _tc_gru_body,
                (jax.ShapeDtypeStruct((n_pad, HID), jnp.float32),
                 jax.ShapeDtypeStruct((n_pad, HID), jnp.float32),
                 jax.ShapeDtypeStruct((NC, n_pad, HALF), jnp.float32)),
                accm, h, hist, _f32(Wi[i]), _f32(Wh[i]), bi2[i], bh2[i],
                _f32(Wm[i + 1]))
        else:
            hist = _tc_call(
                _tc_gru_last_body,
                jax.ShapeDtypeStruct((n_pad, HID), jnp.float32),
                accm, h, hist, _f32(Wi[i]), _f32(Wh[i]), bi2[i], bh2[i])

    scores2d, loss2d = _tc_call(
        _tc_final_body,
        (jax.ShapeDtypeStruct((g_cnt, 1), jnp.float32),
         jax.ShapeDtypeStruct((1, 1), jnp.float32)),
        hist, batch2d, y2d, _f32(W1), b12, _f32(W2), b22)

    return scores2d.reshape(g_cnt), loss2d.reshape(())


# speedup vs baseline: 1.2123x; 1.2123x over previous
import functools

import jax
import jax.numpy as jnp
from jax import lax
from jax.experimental import pallas as pl
from jax.experimental.pallas import tpu as pltpu
from jax.experimental.pallas import tpu_sc as plsc

N_BLOCKS = 3
HEADS = 4
OUT = 32
HID = 128
HALF = HID // 2
NEG_SLOPE = 0.2

NC = 2
NS = 16
LANES = 16
CHUNK = 80
GBLK = 50
NGBLK = 5


def _f32(x):
    return x.astype(jnp.float32)


_GD = lax.GatherDimensionNumbers(offset_dims=(), collapsed_slice_dims=(0,),
                                 start_index_map=(0,))


def _lane_splat(v, idx16):
    return lax.gather(v, idx16[:, None], _GD, slice_sizes=(1,),
                      mode=lax.GatherScatterMode.PROMISE_IN_BOUNDS)


def _tc_a_body(x_ref, wg_ref, asel_ref, adsel_ref, rsel_ref, psel_ref,
               h2_ref, alpha_ref, hinit_ref, dinit_ref):
    x = x_ref[...]
    h0 = jnp.dot(x, wg_ref[...], preferred_element_type=jnp.float32)
    h2_ref[0] = h0[:, :HALF]
    h2_ref[1] = h0[:, HALF:]
    alpha_s = jnp.dot(h0, asel_ref[...], preferred_element_type=jnp.float32)
    alpha_d = jnp.dot(h0, adsel_ref[...], preferred_element_type=jnp.float32)
    alpha_ref[0] = jnp.concatenate([alpha_s[:, 0:2], alpha_d[:, 0:2]], axis=1)
    alpha_ref[1] = jnp.concatenate([alpha_s[:, 2:4], alpha_d[:, 2:4]], axis=1)
    e = alpha_s + alpha_d
    e = jnp.where(e > 0, e, NEG_SLOPE * e)
    w_self = jnp.exp(e)
    w_rep = jnp.dot(w_self, rsel_ref[...], preferred_element_type=jnp.float32)
    hw = h0 * w_rep
    hinit_ref[0] = hw[:, :HALF]
    hinit_ref[1] = hw[:, HALF:]
    hid4 = lax.broadcasted_iota(jnp.int32, w_self.shape, 1)
    d0 = jnp.dot(jnp.where(hid4 < 2, w_self, 0.0), psel_ref[...],
                 preferred_element_type=jnp.float32)
    d1 = jnp.dot(jnp.where(hid4 >= 2, w_self, 0.0), psel_ref[...],
                 preferred_element_type=jnp.float32)
    dinit_ref[0] = d0
    dinit_ref[1] = d1


def _tc_b_body(acch_ref, accd_ref, wm_ref, psum_ref, h_ref, v_ref):
    num = jnp.concatenate([acch_ref[0], acch_ref[1]], axis=1)
    den = accd_ref[0] + accd_ref[1]
    den_rep = jnp.dot(den, psum_ref[...], preferred_element_type=jnp.float32)
    h = num / (den_rep + 1e-16)
    h_ref[...] = h
    v = jnp.dot(h, wm_ref[...], preferred_element_type=jnp.float32)
    v_ref[0] = v[:, :HALF]
    v_ref[1] = v[:, HALF:]


def _gru_math(m, h, wi, wh, bi, bh):
    gi = jnp.dot(m, wi, preferred_element_type=jnp.float32) + bi
    gh = jnp.dot(h, wh, preferred_element_type=jnp.float32) + bh
    ir, iz, ig = gi[:, :HID], gi[:, HID:2 * HID], gi[:, 2 * HID:]
    hr, hz, hg = gh[:, :HID], gh[:, HID:2 * HID], gh[:, 2 * HID:]
    r = jax.nn.sigmoid(ir + hr)
    z = jax.nn.sigmoid(iz + hz)
    g = jnp.tanh(ig + r * hg)
    return (1.0 - z) * g + z * h


def _tc_gru_body(accm_ref, h_ref, hist_ref, wi_ref, wh_ref, bi_ref, bh_ref,
                 wmn_ref, hn_ref, histn_ref, vn_ref):
    m = jnp.concatenate([accm_ref[0], accm_ref[1]], axis=1)
    hn = _gru_math(m, h_ref[...], wi_ref[...], wh_ref[...], bi_ref[...],
                   bh_ref[...])
    hn_ref[...] = hn
    histn_ref[...] = hist_ref[...] + hn
    v = jnp.dot(hn, wmn_ref[...], preferred_element_type=jnp.float32)
    vn_ref[0] = v[:, :HALF]
    vn_ref[1] = v[:, HALF:]


def _tc_gru_last_body(accm_ref, h_ref, hist_ref, wi_ref, wh_ref, bi_ref,
                      bh_ref, histn_ref):
    m = jnp.concatenate([accm_ref[0], accm_ref[1]], axis=1)
    hn = _gru_math(m, h_ref[...], wi_ref[...], wh_ref[...], bi_ref[...],
                   bh_ref[...])
    histn_ref[...] = hist_ref[...] + hn


def _tc_final_body(hist_ref, batch_ref, y_ref, w1_ref, b1_ref, w2_ref, b2_ref,
                   scores_ref, loss_ref):
    n = hist_ref.shape[0]
    g = y_ref.shape[0]
    batch = batch_ref[...]
    gids = lax.broadcasted_iota(jnp.int32, (g, n), 0)
    sel = jnp.where(gids == batch, 1.0, 0.0).astype(jnp.float32)
    pooled = jnp.dot(sel, hist_ref[...], preferred_element_type=jnp.float32)
    z = jnp.maximum(
        jnp.dot(pooled, w1_ref[...], preferred_element_type=jnp.float32)
        + b1_ref[...], 0.0)
    s = jnp.dot(z, w2_ref[...], preferred_element_type=jnp.float32) \
        + b2_ref[...]
    scores_ref[...] = s
    y = y_ref[...]
    lv = jnp.maximum(s, 0.0) - s * y + jnp.log1p(jnp.exp(-jnp.abs(s)))
    loss_ref[...] = jnp.sum(lv).reshape(1, 1) / g


def _tc_call(body, out_shapes, *args):
    return pl.pallas_call(
        body, out_shape=out_shapes,
        compiler_params=pltpu.CompilerParams(
            vmem_limit_bytes=100 * 1024 * 1024))(*args)


def _segsum_sc(v2, src3d, dst3d, n):
    chunks_tile = src3d.shape[1]
    mchunk = src3d.shape[2]
    rows_tile = n // NS
    mesh = plsc.VectorSubcoreMesh(core_axis_name="c", subcore_axis_name="s",
                                  num_cores=NC, num_subcores=NS)

    @functools.partial(
        pl.kernel,
        out_type=jax.ShapeDtypeStruct((NC, n, HALF), jnp.float32),
        mesh=mesh,
        compiler_params=pltpu.CompilerParams(needs_layout_passes=False,
                                             use_tc_tiling_on_sc=False),
        scratch_types=[
            pltpu.VMEM((chunks_tile, mchunk), jnp.int32),
            pltpu.VMEM((chunks_tile, mchunk), jnp.int32),
            pltpu.VMEM((mchunk, HALF), jnp.float32),
            pltpu.VMEM((mchunk, HALF), jnp.float32),
            pltpu.VMEM((mchunk, HALF), jnp.float32),
            pltpu.VMEM((mchunk, HALF), jnp.float32),
            pltpu.VMEM_SHARED((n, HALF), jnp.float32),
            pltpu.SemaphoreType.DMA,
            pltpu.SemaphoreType.DMA,
            pltpu.SemaphoreType.DMA,
            pltpu.SemaphoreType.DMA,
            pltpu.SemaphoreType.DMA,
            pltpu.SemaphoreType.DMA,
            pltpu.SemaphoreType.DMA,
            pltpu.SemaphoreType.DMA,
        ],
    )
    def k(v_hbm, src_hbm, dst_hbm, z_hbm, out_hbm,
          sidx, didx, r0, r1, r2, r3, acc,
          g0, g1, g2, g3, s0, s1, s2, s3):
        rows = [r0, r1, r2, r3]
        gsem = [g0, g1, g2, g3]
        ssem = [s0, s1, s2, s3]
        cid = lax.axis_index("c")
        sid = lax.axis_index("s")
        pltpu.sync_copy(src_hbm.at[sid], sidx)
        pltpu.sync_copy(dst_hbm.at[sid], didx)
        rbase = sid * rows_tile
        pltpu.sync_copy(z_hbm.at[pl.ds(rbase, rows_tile)],
                        acc.at[pl.ds(rbase, rows_tile)])
        plsc.subcore_barrier()

        last = chunks_tile - 1

        def gissue(c, j):
            pltpu.async_copy(v_hbm.at[cid].at[sidx.at[c]], rows[j], gsem[j])

        def gwait(j):
            pltpu.make_async_copy(v_hbm.at[cid].at[sidx.at[0]], rows[j],
                                  gsem[j]).wait()

        def sissue(c, j):
            pltpu.async_copy(rows[j], acc.at[didx.at[c]], ssem[j], add=True)

        def swait(j):
            pltpu.make_async_copy(rows[j], acc.at[didx.at[0]],
                                  ssem[j]).wait()

        gissue(0, 0)
        gissue(1, 1)

        def body(i, _):
            for j in range(4):
                c = 4 * i + j
                jj = (j + 2) % 4

                @pl.when(c >= 2)
                def _():
                    swait(jj)

                gissue(jnp.minimum(c + 2, last), jj)
                gwait(j)
                sissue(c, j)
            return 0

        lax.fori_loop(0, chunks_tile // 4, body, 0)
        swait(2)
        swait(3)
        gwait(0)
        gwait(1)
        plsc.subcore_barrier()
        pltpu.sync_copy(acc.at[pl.ds(rbase, rows_tile)],
                        out_hbm.at[cid, pl.ds(rbase, rows_tile)])

    zeros = jnp.zeros((n, HALF), jnp.float32)
    return k(v2, src3d, dst3d, zeros)


def _gat_edges_sc(h2, alpha2, src4d, dst4d, hinit2, dinit2, n):
    nblk = src4d.shape[1]
    gblk = src4d.shape[2]
    rows_tile = n // NS
    mesh = plsc.VectorSubcoreMesh(core_axis_name="c", subcore_axis_name="s",
                                  num_cores=NC, num_subcores=NS)
    groups = CHUNK // LANES

    @functools.partial(
        pl.kernel,
        out_type=(jax.ShapeDtypeStruct((NC, n, HALF), jnp.float32),
                  jax.ShapeDtypeStruct((NC, n, LANES), jnp.float32)),
        mesh=mesh,
        compiler_params=pltpu.CompilerParams(needs_layout_passes=False,
                                             use_tc_tiling_on_sc=False),
        scratch_types=[
            pltpu.VMEM((n * 4,), jnp.float32),
            pltpu.VMEM((GBLK, CHUNK), jnp.int32),
            pltpu.VMEM((GBLK, CHUNK), jnp.int32),
            pltpu.VMEM((CHUNK, HALF), jnp.float32),
            pltpu.VMEM((CHUNK, HALF), jnp.float32),
            pltpu.VMEM((CHUNK, HALF), jnp.float32),
            pltpu.VMEM((CHUNK, LANES), jnp.float32),
            pltpu.VMEM_SHARED((n, HALF), jnp.float32),
            pltpu.VMEM_SHARED((n, LANES), jnp.float32),
            pltpu.SemaphoreType.DMA,
            pltpu.SemaphoreType.DMA,
        ],
    )
    def k(h_hbm, alpha_hbm, src_hbm, dst_hbm, hinit_hbm, dinit_hbm,
          outh_hbm, outd_hbm,
          alpha_v, sidx, didx, rows_a, rows_b, rows_s, wrow, acch, accd,
          sem_a, sem_b):
        cid = lax.axis_index("c")
        sid = lax.axis_index("s")
        pltpu.sync_copy(alpha_hbm.at[cid], alpha_v)
        rbase = sid * rows_tile
        pltpu.sync_copy(hinit_hbm.at[cid, pl.ds(rbase, rows_tile)],
                        acch.at[pl.ds(rbase, rows_tile)])
        pltpu.sync_copy(dinit_hbm.at[cid, pl.ds(rbase, rows_tile)],
                        accd.at[pl.ds(rbase, rows_tile)])
        zf = jnp.zeros((LANES,), jnp.float32)
        for kk in range(CHUNK):
            wrow[kk, :] = zf
        plsc.subcore_barrier()

        def gather(c, rows, sem):
            return pltpu.async_copy(h_hbm.at[cid].at[sidx.at[c]], rows, sem)

        def scale(c, rows):
            lane = lax.iota(jnp.int32, LANES)
            for g in range(groups):
                el = g * LANES + lane
                s_i = sidx[c, pl.ds(g * LANES, LANES)]
                d_i = didx[c, pl.ds(g * LANES, LANES)]
                ws = []
                for hl in range(2):
                    a_s = plsc.load_gather(alpha_v, [s_i * 4 + hl])
                    a_d = plsc.load_gather(alpha_v, [d_i * 4 + 2 + hl])
                    e = a_s + a_d
                    e = jnp.where(e > 0, e, NEG_SLOPE * e)
                    w = jnp.exp(e)
                    ws.append(w)
                    wcol = jnp.full((LANES,), hl, jnp.int32) + 2 * cid
                    plsc.store_scatter(wrow, [el, wcol], w)
                for k in range(LANES):
                    ek = g * LANES + k
                    wv = wrow[ek, pl.ds(0, LANES)]
                    for hl in range(2):
                        wk = _lane_splat(
                            wv, jnp.full((LANES,), hl, jnp.int32) + 2 * cid)
                        base = hl * OUT
                        rows_s[ek, pl.ds(base, LANES)] = (
                            rows[ek, pl.ds(base, LANES)] * wk)
                        rows_s[ek, pl.ds(base + LANES, LANES)] = (
                            rows[ek, pl.ds(base + LANES, LANES)] * wk)

        def scat(c, rows):
            pltpu.sync_copy(rows_s, acch.at[didx.at[c]], add=True)
            pltpu.sync_copy(wrow, accd.at[didx.at[c]], add=True)

        def blk(b, _):
            pltpu.sync_copy(src_hbm.at[sid, b], sidx)
            pltpu.sync_copy(dst_hbm.at[sid, b], didx)
            gather(0, rows_a, sem_a).wait()

            def body(i, _):
                c = 2 * i
                db = gather(c + 1, rows_b, sem_b)
                scale(c, rows_a)
                scat(c, rows_a)
                da = gather(jnp.minimum(c + 2, gblk - 1), rows_a, sem_a)
                db.wait()
                scale(c + 1, rows_b)
                scat(c + 1, rows_b)
                da.wait()
                return 0

            lax.fori_loop(0, gblk // 2, body, 0)
            return 0

        lax.fori_loop(0, nblk, blk, 0)
        plsc.subcore_barrier()
        pltpu.sync_copy(acch.at[pl.ds(rbase, rows_tile)],
                        outh_hbm.at[cid, pl.ds(rbase, rows_tile)])
        pltpu.sync_copy(accd.at[pl.ds(rbase, rows_tile)],
                        outd_hbm.at[cid, pl.ds(rbase, rows_tile)])

    return k(h2, alpha2, src4d, dst4d, hinit2, dinit2)


def kernel(x, edge_index, batch, y, num_graphs, W_gat, a_src, a_dst, Wm, Wi,
           Wh, bi, bh, W1, b1, W2, b2):
    n = x.shape[0]
    e_cnt = edge_index.shape[1]
    g_cnt = y.shape[0]
    del num_graphs

    eye4 = jnp.eye(HEADS, dtype=jnp.float32)
    asel = jnp.einsum('hk,hg->hkg', _f32(a_src), eye4).reshape(HID, HEADS)
    adsel = jnp.einsum('hk,hg->hkg', _f32(a_dst), eye4).reshape(HID, HEADS)
    rsel = jnp.repeat(eye4, OUT, axis=1).reshape(HEADS, HID)
    psel = jnp.concatenate(
        [eye4, jnp.zeros((HEADS, LANES - HEADS), jnp.float32)], axis=1)
    psum = jnp.concatenate(
        [jnp.repeat(eye4, OUT, axis=1).reshape(HEADS, HID),
         jnp.zeros((LANES - HEADS, HID), jnp.float32)], axis=0)

    n_pad = ((n + 127) // 128) * 128
    chunks_tile = e_cnt // (NS * CHUNK)
    src = edge_index[0].astype(jnp.int32)
    dst = edge_index[1].astype(jnp.int32)
    mchunk = 100
    src3d = src.reshape(NS, e_cnt // (NS * mchunk), mchunk)
    dst3d = dst.reshape(NS, e_cnt // (NS * mchunk), mchunk)
    src4d = src.reshape(NS, NGBLK, GBLK, CHUNK)
    dst4d = dst.reshape(NS, NGBLK, GBLK, CHUNK)
    x_p = jnp.pad(_f32(x), ((0, n_pad - n), (0, 0)))
    batch2d = jnp.pad(batch.astype(jnp.int32), (0, n_pad - n),
                      constant_values=g_cnt).reshape(1, n_pad)
    y2d = _f32(y).reshape(g_cnt, 1)
    bi2 = _f32(bi).reshape(N_BLOCKS, 1, 3 * HID)
    bh2 = _f32(bh).reshape(N_BLOCKS, 1, 3 * HID)
    b12 = _f32(b1).reshape(1, 64)
    b22 = _f32(b2).reshape(1, 1)

    h2, alpha2, hinit2, dinit2 = _tc_call(
        _tc_a_body,
        (jax.ShapeDtypeStruct((NC, n_pad, HALF), jnp.float32),
         jax.ShapeDtypeStruct((NC, n_pad, HEADS), jnp.float32),
         jax.ShapeDtypeStruct((NC, n_pad, HALF), jnp.float32),
         jax.ShapeDtypeStruct((NC, n_pad, LANES), jnp.float32)),
        x_p, _f32(W_gat), asel, adsel, rsel, psel)

    acch, accd = _gat_edges_sc(h2, alpha2.reshape(NC, n_pad * HEADS), src4d,
                               dst4d, hinit2, dinit2, n_pad)

    h, v2 = _tc_call(
        _tc_b_body,
        (jax.ShapeDtypeStruct((n_pad, HID), jnp.float32),
         jax.ShapeDtypeStruct((NC, n_pad, HALF), jnp.float32)),
        acch, accd, _f32(Wm[0]), psum)
    hist = h

    for i in range(N_BLOCKS):
        accm = _segsum_sc(v2, src3d, dst3d, n_pad)
        if i + 1 < N_BLOCKS:
            h, hist, v2 = _tc_call(
                _tc_gru_body,
                (jax.ShapeDtypeStruct((n_pad, HID), jnp.float32),
                 jax.ShapeDtypeStruct((n_pad, HID), jnp.float32),
                 jax.ShapeDtypeStruct((NC, n_pad, HALF), jnp.float32)),
                accm, h, hist, _f32(Wi[i]), _f32(Wh[i]), bi2[i], bh2[i],
                _f32(Wm[i + 1]))
        else:
            hist = _tc_call(
                _tc_gru_last_body,
                jax.ShapeDtypeStruct((n_pad, HID), jnp.float32),
                accm, h, hist, _f32(Wi[i]), _f32(Wh[i]), bi2[i], bh2[i])

    scores2d, loss2d = _tc_call(
        _tc_final_body,
        (jax.ShapeDtypeStruct((g_cnt, 1), jnp.float32),
         jax.ShapeDtypeStruct((1, 1), jnp.float32)),
        hist, batch2d, y2d, _f32(W1), b12, _f32(W2), b22)

    return scores2d.reshape(g_cnt), loss2d.reshape(())


# speedup vs baseline: 1.2810x; 1.0566x over previous
import functools

import jax
import jax.numpy as jnp
from jax import lax
from jax.experimental import pallas as pl
from jax.experimental.pallas import tpu as pltpu
from jax.experimental.pallas import tpu_sc as plsc

N_BLOCKS = 3
HEADS = 4
OUT = 32
HID = 128
HALF = HID // 2
NEG_SLOPE = 0.2

NC = 2
NS = 16
LANES = 16
CHUNK = 80
GBLK = 50
NGBLK = 5


def _f32(x):
    return x.astype(jnp.float32)


_GD = lax.GatherDimensionNumbers(offset_dims=(), collapsed_slice_dims=(0,),
                                 start_index_map=(0,))


def _lane_splat(v, idx16):
    return lax.gather(v, idx16[:, None], _GD, slice_sizes=(1,),
                      mode=lax.GatherScatterMode.PROMISE_IN_BOUNDS)


def _tc_a_body(x_ref, wg_ref, asel_ref, adsel_ref, rsel_ref, psel_ref,
               h2_ref, alpha_ref, hinit_ref, dinit_ref):
    x = x_ref[...]
    h0 = jnp.dot(x, wg_ref[...], preferred_element_type=jnp.float32)
    h2_ref[0] = h0[:, :HALF]
    h2_ref[1] = h0[:, HALF:]
    alpha_s = jnp.dot(h0, asel_ref[...], preferred_element_type=jnp.float32)
    alpha_d = jnp.dot(h0, adsel_ref[...], preferred_element_type=jnp.float32)
    alpha_ref[0] = jnp.concatenate([alpha_s[:, 0:2], alpha_d[:, 0:2]], axis=1)
    alpha_ref[1] = jnp.concatenate([alpha_s[:, 2:4], alpha_d[:, 2:4]], axis=1)
    e = alpha_s + alpha_d
    e = jnp.where(e > 0, e, NEG_SLOPE * e)
    w_self = jnp.exp(e)
    w_rep = jnp.dot(w_self, rsel_ref[...], preferred_element_type=jnp.float32)
    hw = h0 * w_rep
    hinit_ref[0] = hw[:, :HALF]
    hinit_ref[1] = hw[:, HALF:]
    hid4 = lax.broadcasted_iota(jnp.int32, w_self.shape, 1)
    d0 = jnp.dot(jnp.where(hid4 < 2, w_self, 0.0), psel_ref[...],
                 preferred_element_type=jnp.float32)
    d1 = jnp.dot(jnp.where(hid4 >= 2, w_self, 0.0), psel_ref[...],
                 preferred_element_type=jnp.float32)
    dinit_ref[0] = d0
    dinit_ref[1] = d1


def _tc_b_body(acch_ref, accd_ref, wm_ref, psum_ref, h_ref, v_ref):
    num = jnp.concatenate([acch_ref[0], acch_ref[1]], axis=1)
    den = accd_ref[0] + accd_ref[1]
    den_rep = jnp.dot(den, psum_ref[...], preferred_element_type=jnp.float32)
    h = num / (den_rep + 1e-16)
    h_ref[...] = h
    v = jnp.dot(h, wm_ref[...], preferred_element_type=jnp.float32)
    v_ref[0] = v[:, :HALF]
    v_ref[1] = v[:, HALF:]


def _gru_math(m, h, wi, wh, bi, bh):
    gi = jnp.dot(m, wi, preferred_element_type=jnp.float32) + bi
    gh = jnp.dot(h, wh, preferred_element_type=jnp.float32) + bh
    ir, iz, ig = gi[:, :HID], gi[:, HID:2 * HID], gi[:, 2 * HID:]
    hr, hz, hg = gh[:, :HID], gh[:, HID:2 * HID], gh[:, 2 * HID:]
    r = jax.nn.sigmoid(ir + hr)
    z = jax.nn.sigmoid(iz + hz)
    g = jnp.tanh(ig + r * hg)
    return (1.0 - z) * g + z * h


def _tc_gru_body(accm_ref, h_ref, hist_ref, wi_ref, wh_ref, bi_ref, bh_ref,
                 wmn_ref, hn_ref, histn_ref, vn_ref):
    m = jnp.concatenate([accm_ref[0], accm_ref[1]], axis=1)
    hn = _gru_math(m, h_ref[...], wi_ref[...], wh_ref[...], bi_ref[...],
                   bh_ref[...])
    hn_ref[...] = hn
    histn_ref[...] = hist_ref[...] + hn
    v = jnp.dot(hn, wmn_ref[...], preferred_element_type=jnp.float32)
    vn_ref[0] = v[:, :HALF]
    vn_ref[1] = v[:, HALF:]


def _tc_gru_last_body(accm_ref, h_ref, hist_ref, wi_ref, wh_ref, bi_ref,
                      bh_ref, histn_ref):
    m = jnp.concatenate([accm_ref[0], accm_ref[1]], axis=1)
    hn = _gru_math(m, h_ref[...], wi_ref[...], wh_ref[...], bi_ref[...],
                   bh_ref[...])
    histn_ref[...] = hist_ref[...] + hn


def _tc_final_body(hist_ref, batch_ref, y_ref, w1_ref, b1_ref, w2_ref, b2_ref,
                   scores_ref, loss_ref):
    n = hist_ref.shape[0]
    g = y_ref.shape[0]
    batch = batch_ref[...]
    gids = lax.broadcasted_iota(jnp.int32, (g, n), 0)
    sel = jnp.where(gids == batch, 1.0, 0.0).astype(jnp.float32)
    pooled = jnp.dot(sel, hist_ref[...], preferred_element_type=jnp.float32)
    z = jnp.maximum(
        jnp.dot(pooled, w1_ref[...], preferred_element_type=jnp.float32)
        + b1_ref[...], 0.0)
    s = jnp.dot(z, w2_ref[...], preferred_element_type=jnp.float32) \
        + b2_ref[...]
    scores_ref[...] = s
    y = y_ref[...]
    lv = jnp.maximum(s, 0.0) - s * y + jnp.log1p(jnp.exp(-jnp.abs(s)))
    loss_ref[...] = jnp.sum(lv).reshape(1, 1) / g


def _tc_call(body, out_shapes, *args):
    return pl.pallas_call(
        body, out_shape=out_shapes,
        compiler_params=pltpu.CompilerParams(
            vmem_limit_bytes=100 * 1024 * 1024))(*args)


def _segsum_sc(v2, src3d, dst3d, n):
    chunks_tile = src3d.shape[1]
    mchunk = src3d.shape[2]
    rows_tile = n // NS
    mesh = plsc.VectorSubcoreMesh(core_axis_name="c", subcore_axis_name="s",
                                  num_cores=NC, num_subcores=NS)

    @functools.partial(
        pl.kernel,
        out_type=jax.ShapeDtypeStruct((NC, n, HALF), jnp.float32),
        mesh=mesh,
        compiler_params=pltpu.CompilerParams(needs_layout_passes=False,
                                             use_tc_tiling_on_sc=False),
        scratch_types=[
            pltpu.VMEM((chunks_tile, mchunk), jnp.int32),
            pltpu.VMEM((chunks_tile, mchunk), jnp.int32),
            pltpu.VMEM((mchunk, HALF), jnp.float32),
            pltpu.VMEM((mchunk, HALF), jnp.float32),
            pltpu.VMEM((mchunk, HALF), jnp.float32),
            pltpu.VMEM((mchunk, HALF), jnp.float32),
            pltpu.VMEM_SHARED((n, HALF), jnp.float32),
            pltpu.SemaphoreType.DMA,
            pltpu.SemaphoreType.DMA,
            pltpu.SemaphoreType.DMA,
            pltpu.SemaphoreType.DMA,
            pltpu.SemaphoreType.DMA,
            pltpu.SemaphoreType.DMA,
            pltpu.SemaphoreType.DMA,
            pltpu.SemaphoreType.DMA,
        ],
    )
    def k(v_hbm, src_hbm, dst_hbm, z_hbm, out_hbm,
          sidx, didx, r0, r1, r2, r3, acc,
          g0, g1, g2, g3, s0, s1, s2, s3):
        rows = [r0, r1, r2, r3]
        gsem = [g0, g1, g2, g3]
        ssem = [s0, s1, s2, s3]
        cid = lax.axis_index("c")
        sid = lax.axis_index("s")
        pltpu.sync_copy(src_hbm.at[sid], sidx)
        pltpu.sync_copy(dst_hbm.at[sid], didx)
        rbase = sid * rows_tile
        pltpu.sync_copy(z_hbm.at[pl.ds(rbase, rows_tile)],
                        acc.at[pl.ds(rbase, rows_tile)])
        plsc.subcore_barrier()

        last = chunks_tile - 1

        def gissue(c, j):
            pltpu.async_copy(v_hbm.at[cid].at[sidx.at[c]], rows[j], gsem[j])

        def gwait(j):
            pltpu.make_async_copy(v_hbm.at[cid].at[sidx.at[0]], rows[j],
                                  gsem[j]).wait()

        def sissue(c, j):
            pltpu.async_copy(rows[j], acc.at[didx.at[c]], ssem[j], add=True)

        def swait(j):
            pltpu.make_async_copy(rows[j], acc.at[didx.at[0]],
                                  ssem[j]).wait()

        gissue(0, 0)
        gissue(1, 1)

        def body(i, _):
            for j in range(4):
                c = 4 * i + j
                jj = (j + 2) % 4

                @pl.when(c >= 2)
                def _():
                    swait(jj)

                gissue(jnp.minimum(c + 2, last), jj)
                gwait(j)
                sissue(c, j)
            return 0

        lax.fori_loop(0, chunks_tile // 4, body, 0)
        swait(2)
        swait(3)
        gwait(0)
        gwait(1)
        plsc.subcore_barrier()
        pltpu.sync_copy(acc.at[pl.ds(rbase, rows_tile)],
                        out_hbm.at[cid, pl.ds(rbase, rows_tile)])

    zeros = jnp.zeros((n, HALF), jnp.float32)
    return k(v2, src3d, dst3d, zeros)


def _gat_edges_sc(h2, alpha2, src4d, dst4d, hinit2, dinit2, n):
    nblk = src4d.shape[1]
    gblk = src4d.shape[2]
    rows_tile = n // NS
    mesh = plsc.VectorSubcoreMesh(core_axis_name="c", subcore_axis_name="s",
                                  num_cores=NC, num_subcores=NS)
    groups = CHUNK // LANES

    @functools.partial(
        pl.kernel,
        out_type=(jax.ShapeDtypeStruct((NC, n, HALF), jnp.float32),
                  jax.ShapeDtypeStruct((NC, n, LANES), jnp.float32)),
        mesh=mesh,
        compiler_params=pltpu.CompilerParams(needs_layout_passes=False,
                                             use_tc_tiling_on_sc=False),
        scratch_types=[
            pltpu.VMEM((n * 4,), jnp.float32),
            pltpu.VMEM((GBLK, CHUNK), jnp.int32),
            pltpu.VMEM((GBLK, CHUNK), jnp.int32),
            pltpu.VMEM((CHUNK, HALF), jnp.float32),
            pltpu.VMEM((CHUNK, HALF), jnp.float32),
            pltpu.VMEM((CHUNK, HALF), jnp.float32),
            pltpu.VMEM((CHUNK, HALF), jnp.float32),
            pltpu.VMEM((CHUNK, LANES), jnp.float32),
            pltpu.VMEM((CHUNK, LANES), jnp.float32),
            pltpu.VMEM_SHARED((n, HALF), jnp.float32),
            pltpu.VMEM_SHARED((n, LANES), jnp.float32),
            pltpu.SemaphoreType.DMA,
            pltpu.SemaphoreType.DMA,
            pltpu.SemaphoreType.DMA,
            pltpu.SemaphoreType.DMA,
        ],
    )
    def k(h_hbm, alpha_hbm, src_hbm, dst_hbm, hinit_hbm, dinit_hbm,
          outh_hbm, outd_hbm,
          alpha_v, sidx, didx, rows_a, rows_b, rs_a, rs_b, wr_a, wr_b,
          acch, accd, sem_a, sem_b, sca, scb):
        cid = lax.axis_index("c")
        sid = lax.axis_index("s")
        pltpu.sync_copy(alpha_hbm.at[cid], alpha_v)
        rbase = sid * rows_tile
        pltpu.sync_copy(hinit_hbm.at[cid, pl.ds(rbase, rows_tile)],
                        acch.at[pl.ds(rbase, rows_tile)])
        pltpu.sync_copy(dinit_hbm.at[cid, pl.ds(rbase, rows_tile)],
                        accd.at[pl.ds(rbase, rows_tile)])
        zf = jnp.zeros((LANES,), jnp.float32)
        for kk in range(CHUNK):
            wr_a[kk, :] = zf
            wr_b[kk, :] = zf
        plsc.subcore_barrier()

        def gather(c, rows, sem):
            return pltpu.async_copy(h_hbm.at[cid].at[sidx.at[c]], rows, sem)

        def scale(c, rows, rs, wrow):
            lane = lax.iota(jnp.int32, LANES)
            for g in range(groups):
                el = g * LANES + lane
                s_i = sidx[c, pl.ds(g * LANES, LANES)]
                d_i = didx[c, pl.ds(g * LANES, LANES)]
                for hl in range(2):
                    a_s = plsc.load_gather(alpha_v, [s_i * 4 + hl])
                    a_d = plsc.load_gather(alpha_v, [d_i * 4 + 2 + hl])
                    e = a_s + a_d
                    e = jnp.where(e > 0, e, NEG_SLOPE * e)
                    w = jnp.exp(e)
                    wcol = jnp.full((LANES,), hl, jnp.int32) + 2 * cid
                    plsc.store_scatter(wrow, [el, wcol], w)
                for k in range(LANES):
                    ek = g * LANES + k
                    wv = wrow[ek, pl.ds(0, LANES)]
                    for hl in range(2):
                        wk = _lane_splat(
                            wv, jnp.full((LANES,), hl, jnp.int32) + 2 * cid)
                        base = hl * OUT
                        rs[ek, pl.ds(base, LANES)] = (
                            rows[ek, pl.ds(base, LANES)] * wk)
                        rs[ek, pl.ds(base + LANES, LANES)] = (
                            rows[ek, pl.ds(base + LANES, LANES)] * wk)

        def sissue(c, rs, wrow, sem):
            pltpu.async_copy(rs, acch.at[didx.at[c]], sem, add=True)
            pltpu.async_copy(wrow, accd.at[didx.at[c]], sem, add=True)

        def swait(rs, wrow, sem):
            pltpu.make_async_copy(rs, acch.at[didx.at[0]], sem).wait()
            pltpu.make_async_copy(wrow, accd.at[didx.at[0]], sem).wait()

        def blk(b, _):
            pltpu.sync_copy(src_hbm.at[sid, b], sidx)
            pltpu.sync_copy(dst_hbm.at[sid, b], didx)
            gather(0, rows_a, sem_a).wait()

            def body(i, _):
                c = 2 * i
                db = gather(c + 1, rows_b, sem_b)

                @pl.when((b > 0) | (i > 0))
                def _():
                    swait(rs_a, wr_a, sca)

                scale(c, rows_a, rs_a, wr_a)
                sissue(c, rs_a, wr_a, sca)
                da = gather(jnp.minimum(c + 2, GBLK - 1), rows_a, sem_a)
                db.wait()

                @pl.when((b > 0) | (i > 0))
                def _():
                    swait(rs_b, wr_b, scb)

                scale(c + 1, rows_b, rs_b, wr_b)
                sissue(c + 1, rs_b, wr_b, scb)
                da.wait()
                return 0

            lax.fori_loop(0, GBLK // 2, body, 0)
            return 0

        lax.fori_loop(0, nblk, blk, 0)
        swait(rs_a, wr_a, sca)
        swait(rs_b, wr_b, scb)
        plsc.subcore_barrier()
        pltpu.sync_copy(acch.at[pl.ds(rbase, rows_tile)],
                        outh_hbm.at[cid, pl.ds(rbase, rows_tile)])
        pltpu.sync_copy(accd.at[pl.ds(rbase, rows_tile)],
                        outd_hbm.at[cid, pl.ds(rbase, rows_tile)])

    return k(h2, alpha2, src4d, dst4d, hinit2, dinit2)


def kernel(x, edge_index, batch, y, num_graphs, W_gat, a_src, a_dst, Wm, Wi,
           Wh, bi, bh, W1, b1, W2, b2):
    n = x.shape[0]
    e_cnt = edge_index.shape[1]
    g_cnt = y.shape[0]
    del num_graphs

    eye4 = jnp.eye(HEADS, dtype=jnp.float32)
    asel = jnp.einsum('hk,hg->hkg', _f32(a_src), eye4).reshape(HID, HEADS)
    adsel = jnp.einsum('hk,hg->hkg', _f32(a_dst), eye4).reshape(HID, HEADS)
    rsel = jnp.repeat(eye4, OUT, axis=1).reshape(HEADS, HID)
    psel = jnp.concatenate(
        [eye4, jnp.zeros((HEADS, LANES - HEADS), jnp.float32)], axis=1)
    psum = jnp.concatenate(
        [jnp.repeat(eye4, OUT, axis=1).reshape(HEADS, HID),
         jnp.zeros((LANES - HEADS, HID), jnp.float32)], axis=0)

    n_pad = ((n + 127) // 128) * 128
    chunks_tile = e_cnt // (NS * CHUNK)
    src = edge_index[0].astype(jnp.int32)
    dst = edge_index[1].astype(jnp.int32)
    mchunk = 100
    src3d = src.reshape(NS, e_cnt // (NS * mchunk), mchunk)
    dst3d = dst.reshape(NS, e_cnt // (NS * mchunk), mchunk)
    src4d = src.reshape(NS, NGBLK, GBLK, CHUNK)
    dst4d = dst.reshape(NS, NGBLK, GBLK, CHUNK)
    x_p = jnp.pad(_f32(x), ((0, n_pad - n), (0, 0)))
    batch2d = jnp.pad(batch.astype(jnp.int32), (0, n_pad - n),
                      constant_values=g_cnt).reshape(1, n_pad)
    y2d = _f32(y).reshape(g_cnt, 1)
    bi2 = _f32(bi).reshape(N_BLOCKS, 1, 3 * HID)
    bh2 = _f32(bh).reshape(N_BLOCKS, 1, 3 * HID)
    b12 = _f32(b1).reshape(1, 64)
    b22 = _f32(b2).reshape(1, 1)

    h2, alpha2, hinit2, dinit2 = _tc_call(
        _tc_a_body,
        (jax.ShapeDtypeStruct((NC, n_pad, HALF), jnp.float32),
         jax.ShapeDtypeStruct((NC, n_pad, HEADS), jnp.float32),
         jax.ShapeDtypeStruct((NC, n_pad, HALF), jnp.float32),
         jax.ShapeDtypeStruct((NC, n_pad, LANES), jnp.float32)),
        x_p, _f32(W_gat), asel, adsel, rsel, psel)

    acch, accd = _gat_edges_sc(h2, alpha2.reshape(NC, n_pad * HEADS), src4d,
                               dst4d, hinit2, dinit2, n_pad)

    h, v2 = _tc_call(
        _tc_b_body,
        (jax.ShapeDtypeStruct((n_pad, HID), jnp.float32),
         jax.ShapeDtypeStruct((NC, n_pad, HALF), jnp.float32)),
        acch, accd, _f32(Wm[0]), psum)
    hist = h

    for i in range(N_BLOCKS):
        accm = _segsum_sc(v2, src3d, dst3d, n_pad)
        if i + 1 < N_BLOCKS:
            h, hist, v2 = _tc_call(
                _tc_gru_body,
                (jax.ShapeDtypeStruct((n_pad, HID), jnp.float32),
                 jax.ShapeDtypeStruct((n_pad, HID), jnp.float32),
                 jax.ShapeDtypeStruct((NC, n_pad, HALF), jnp.float32)),
                accm, h, hist, _f32(Wi[i]), _f32(Wh[i]), bi2[i], bh2[i],
                _f32(Wm[i + 1]))
        else:
            hist = _tc_call(
                _tc_gru_last_body,
                jax.ShapeDtypeStruct((n_pad, HID), jnp.float32),
                accm, h, hist, _f32(Wi[i]), _f32(Wh[i]), bi2[i], bh2[i])

    scores2d, loss2d = _tc_call(
        _tc_final_body,
        (jax.ShapeDtypeStruct((g_cnt, 1), jnp.float32),
         jax.ShapeDtypeStruct((1, 1), jnp.float32)),
        hist, batch2d, y2d, _f32(W1), b12, _f32(W2), b22)

    return scores2d.reshape(g_cnt), loss2d.reshape(())


# speedup vs baseline: 1.2886x; 1.0059x over previous
import functools

import jax
import jax.numpy as jnp
from jax import lax
from jax.experimental import pallas as pl
from jax.experimental.pallas import tpu as pltpu
from jax.experimental.pallas import tpu_sc as plsc

N_BLOCKS = 3
HEADS = 4
OUT = 32
HID = 128
HALF = HID // 2
NEG_SLOPE = 0.2

NC = 2
NS = 16
LANES = 16
CHUNK = 80
GBLK = 50
NGBLK = 5


def _f32(x):
    return x.astype(jnp.float32)


_GD = lax.GatherDimensionNumbers(offset_dims=(), collapsed_slice_dims=(0,),
                                 start_index_map=(0,))


def _lane_splat(v, idx16):
    return lax.gather(v, idx16[:, None], _GD, slice_sizes=(1,),
                      mode=lax.GatherScatterMode.PROMISE_IN_BOUNDS)


def _tc_a_body(x_ref, wg_ref, asel_ref, adsel_ref, rsel_ref, psel_ref,
               h2_ref, alpha_ref, hinit_ref, dinit_ref):
    x = x_ref[...]
    h0 = jnp.dot(x, wg_ref[...], preferred_element_type=jnp.float32)
    h2_ref[0] = h0[:, :HALF]
    h2_ref[1] = h0[:, HALF:]
    alpha_s = jnp.dot(h0, asel_ref[...], preferred_element_type=jnp.float32)
    alpha_d = jnp.dot(h0, adsel_ref[...], preferred_element_type=jnp.float32)
    alpha_ref[0] = jnp.concatenate([alpha_s[:, 0:2], alpha_d[:, 0:2]], axis=1)
    alpha_ref[1] = jnp.concatenate([alpha_s[:, 2:4], alpha_d[:, 2:4]], axis=1)
    e = alpha_s + alpha_d
    e = jnp.where(e > 0, e, NEG_SLOPE * e)
    w_self = jnp.exp(e)
    w_rep = jnp.dot(w_self, rsel_ref[...], preferred_element_type=jnp.float32)
    hw = h0 * w_rep
    hinit_ref[0] = hw[:, :HALF]
    hinit_ref[1] = hw[:, HALF:]
    hid4 = lax.broadcasted_iota(jnp.int32, w_self.shape, 1)
    d0 = jnp.dot(jnp.where(hid4 < 2, w_self, 0.0), psel_ref[...],
                 preferred_element_type=jnp.float32)
    d1 = jnp.dot(jnp.where(hid4 >= 2, w_self, 0.0), psel_ref[...],
                 preferred_element_type=jnp.float32)
    dinit_ref[0] = d0
    dinit_ref[1] = d1


def _tc_b_body(acch_ref, accd_ref, wm_ref, psum_ref, h_ref, v_ref):
    num = jnp.concatenate([acch_ref[0], acch_ref[1]], axis=1)
    den = accd_ref[0] + accd_ref[1]
    den_rep = jnp.dot(den, psum_ref[...], preferred_element_type=jnp.float32)
    h = num / (den_rep + 1e-16)
    h_ref[...] = h
    v = jnp.dot(h, wm_ref[...], preferred_element_type=jnp.float32)
    v_ref[0] = v[:, :HALF]
    v_ref[1] = v[:, HALF:]


def _gru_math(m, h, wi, wh, bi, bh):
    gi = jnp.dot(m, wi, preferred_element_type=jnp.float32) + bi
    gh = jnp.dot(h, wh, preferred_element_type=jnp.float32) + bh
    ir, iz, ig = gi[:, :HID], gi[:, HID:2 * HID], gi[:, 2 * HID:]
    hr, hz, hg = gh[:, :HID], gh[:, HID:2 * HID], gh[:, 2 * HID:]
    r = jax.nn.sigmoid(ir + hr)
    z = jax.nn.sigmoid(iz + hz)
    g = jnp.tanh(ig + r * hg)
    return (1.0 - z) * g + z * h


def _tc_gru_body(accm_ref, h_ref, hist_ref, wi_ref, wh_ref, bi_ref, bh_ref,
                 wmn_ref, hn_ref, histn_ref, vn_ref):
    m = jnp.concatenate([accm_ref[0], accm_ref[1]], axis=1)
    hn = _gru_math(m, h_ref[...], wi_ref[...], wh_ref[...], bi_ref[...],
                   bh_ref[...])
    hn_ref[...] = hn
    histn_ref[...] = hist_ref[...] + hn
    v = jnp.dot(hn, wmn_ref[...], preferred_element_type=jnp.float32)
    vn_ref[0] = v[:, :HALF]
    vn_ref[1] = v[:, HALF:]


def _tc_gru_last_body(accm_ref, h_ref, hist_ref, wi_ref, wh_ref, bi_ref,
                      bh_ref, batch_ref, y_ref, w1_ref, b1_ref, w2_ref,
                      b2_ref, scores_ref, loss_ref):
    m = jnp.concatenate([accm_ref[0], accm_ref[1]], axis=1)
    hn = _gru_math(m, h_ref[...], wi_ref[...], wh_ref[...], bi_ref[...],
                   bh_ref[...])
    hist = hist_ref[...] + hn
    n = hist.shape[0]
    g = y_ref.shape[0]
    batch = batch_ref[...]
    gids = lax.broadcasted_iota(jnp.int32, (g, n), 0)
    sel = jnp.where(gids == batch, 1.0, 0.0).astype(jnp.float32)
    pooled = jnp.dot(sel, hist, preferred_element_type=jnp.float32)
    z = jnp.maximum(
        jnp.dot(pooled, w1_ref[...], preferred_element_type=jnp.float32)
        + b1_ref[...], 0.0)
    s = jnp.dot(z, w2_ref[...], preferred_element_type=jnp.float32) \
        + b2_ref[...]
    scores_ref[...] = s
    y = y_ref[...]
    lv = jnp.maximum(s, 0.0) - s * y + jnp.log1p(jnp.exp(-jnp.abs(s)))
    loss_ref[...] = jnp.sum(lv).reshape(1, 1) / g


def _tc_final_body(hist_ref, batch_ref, y_ref, w1_ref, b1_ref, w2_ref, b2_ref,
                   scores_ref, loss_ref):
    n = hist_ref.shape[0]
    g = y_ref.shape[0]
    batch = batch_ref[...]
    gids = lax.broadcasted_iota(jnp.int32, (g, n), 0)
    sel = jnp.where(gids == batch, 1.0, 0.0).astype(jnp.float32)
    pooled = jnp.dot(sel, hist_ref[...], preferred_element_type=jnp.float32)
    z = jnp.maximum(
        jnp.dot(pooled, w1_ref[...], preferred_element_type=jnp.float32)
        + b1_ref[...], 0.0)
    s = jnp.dot(z, w2_ref[...], preferred_element_type=jnp.float32) \
        + b2_ref[...]
    scores_ref[...] = s
    y = y_ref[...]
    lv = jnp.maximum(s, 0.0) - s * y + jnp.log1p(jnp.exp(-jnp.abs(s)))
    loss_ref[...] = jnp.sum(lv).reshape(1, 1) / g


def _tc_call(body, out_shapes, *args):
    return pl.pallas_call(
        body, out_shape=out_shapes,
        compiler_params=pltpu.CompilerParams(
            vmem_limit_bytes=100 * 1024 * 1024))(*args)


def _segsum_sc(v2, src3d, dst3d, n):
    chunks_tile = src3d.shape[1]
    mchunk = src3d.shape[2]
    rows_tile = n // NS
    mesh = plsc.VectorSubcoreMesh(core_axis_name="c", subcore_axis_name="s",
                                  num_cores=NC, num_subcores=NS)

    @functools.partial(
        pl.kernel,
        out_type=jax.ShapeDtypeStruct((NC, n, HALF), jnp.float32),
        mesh=mesh,
        compiler_params=pltpu.CompilerParams(needs_layout_passes=False,
                                             use_tc_tiling_on_sc=False),
        scratch_types=[
            pltpu.VMEM((chunks_tile, mchunk), jnp.int32),
            pltpu.VMEM((chunks_tile, mchunk), jnp.int32),
            pltpu.VMEM((mchunk, HALF), jnp.float32),
            pltpu.VMEM((mchunk, HALF), jnp.float32),
            pltpu.VMEM((mchunk, HALF), jnp.float32),
            pltpu.VMEM((mchunk, HALF), jnp.float32),
            pltpu.VMEM_SHARED((n, HALF), jnp.float32),
            pltpu.SemaphoreType.DMA,
            pltpu.SemaphoreType.DMA,
            pltpu.SemaphoreType.DMA,
            pltpu.SemaphoreType.DMA,
            pltpu.SemaphoreType.DMA,
            pltpu.SemaphoreType.DMA,
            pltpu.SemaphoreType.DMA,
            pltpu.SemaphoreType.DMA,
        ],
    )
    def k(v_hbm, src_hbm, dst_hbm, z_hbm, out_hbm,
          sidx, didx, r0, r1, r2, r3, acc,
          g0, g1, g2, g3, s0, s1, s2, s3):
        rows = [r0, r1, r2, r3]
        gsem = [g0, g1, g2, g3]
        ssem = [s0, s1, s2, s3]
        cid = lax.axis_index("c")
        sid = lax.axis_index("s")
        pltpu.sync_copy(src_hbm.at[sid], sidx)
        pltpu.sync_copy(dst_hbm.at[sid], didx)
        rbase = sid * rows_tile
        pltpu.sync_copy(z_hbm.at[pl.ds(rbase, rows_tile)],
                        acc.at[pl.ds(rbase, rows_tile)])
        plsc.subcore_barrier()

        last = chunks_tile - 1

        def gissue(c, j):
            pltpu.async_copy(v_hbm.at[cid].at[sidx.at[c]], rows[j], gsem[j])

        def gwait(j):
            pltpu.make_async_copy(v_hbm.at[cid].at[sidx.at[0]], rows[j],
                                  gsem[j]).wait()

        def sissue(c, j):
            pltpu.async_copy(rows[j], acc.at[didx.at[c]], ssem[j], add=True)

        def swait(j):
            pltpu.make_async_copy(rows[j], acc.at[didx.at[0]],
                                  ssem[j]).wait()

        gissue(0, 0)
        gissue(1, 1)

        def body(i, _):
            for j in range(4):
                c = 4 * i + j
                jj = (j + 2) % 4

                @pl.when(c >= 2)
                def _():
                    swait(jj)

                gissue(jnp.minimum(c + 2, last), jj)
                gwait(j)
                sissue(c, j)
            return 0

        lax.fori_loop(0, chunks_tile // 4, body, 0)
        swait(2)
        swait(3)
        gwait(0)
        gwait(1)
        plsc.subcore_barrier()
        pltpu.sync_copy(acc.at[pl.ds(rbase, rows_tile)],
                        out_hbm.at[cid, pl.ds(rbase, rows_tile)])

    zeros = jnp.zeros((n, HALF), jnp.float32)
    return k(v2, src3d, dst3d, zeros)


def _gat_edges_sc(h2, alpha2, src4d, dst4d, hinit2, dinit2, n):
    nblk = src4d.shape[1]
    gblk = src4d.shape[2]
    rows_tile = n // NS
    mesh = plsc.VectorSubcoreMesh(core_axis_name="c", subcore_axis_name="s",
                                  num_cores=NC, num_subcores=NS)
    groups = CHUNK // LANES

    @functools.partial(
        pl.kernel,
        out_type=(jax.ShapeDtypeStruct((NC, n, HALF), jnp.float32),
                  jax.ShapeDtypeStruct((NC, n, LANES), jnp.float32)),
        mesh=mesh,
        compiler_params=pltpu.CompilerParams(needs_layout_passes=False,
                                             use_tc_tiling_on_sc=False),
        scratch_types=[
            pltpu.VMEM((n * 4,), jnp.float32),
            pltpu.VMEM((GBLK, CHUNK), jnp.int32),
            pltpu.VMEM((GBLK, CHUNK), jnp.int32),
            pltpu.VMEM((CHUNK, HALF), jnp.float32),
            pltpu.VMEM((CHUNK, HALF), jnp.float32),
            pltpu.VMEM((CHUNK, HALF), jnp.float32),
            pltpu.VMEM((CHUNK, HALF), jnp.float32),
            pltpu.VMEM((CHUNK, LANES), jnp.float32),
            pltpu.VMEM((CHUNK, LANES), jnp.float32),
            pltpu.VMEM_SHARED((n, HALF), jnp.float32),
            pltpu.VMEM_SHARED((n, LANES), jnp.float32),
            pltpu.SemaphoreType.DMA,
            pltpu.SemaphoreType.DMA,
            pltpu.SemaphoreType.DMA,
            pltpu.SemaphoreType.DMA,
        ],
    )
    def k(h_hbm, alpha_hbm, src_hbm, dst_hbm, hinit_hbm, dinit_hbm,
          outh_hbm, outd_hbm,
          alpha_v, sidx, didx, rows_a, rows_b, rs_a, rs_b, wr_a, wr_b,
          acch, accd, sem_a, sem_b, sca, scb):
        cid = lax.axis_index("c")
        sid = lax.axis_index("s")
        pltpu.sync_copy(alpha_hbm.at[cid], alpha_v)
        rbase = sid * rows_tile
        pltpu.sync_copy(hinit_hbm.at[cid, pl.ds(rbase, rows_tile)],
                        acch.at[pl.ds(rbase, rows_tile)])
        pltpu.sync_copy(dinit_hbm.at[cid, pl.ds(rbase, rows_tile)],
                        accd.at[pl.ds(rbase, rows_tile)])
        zf = jnp.zeros((LANES,), jnp.float32)
        for kk in range(CHUNK):
            wr_a[kk, :] = zf
            wr_b[kk, :] = zf
        plsc.subcore_barrier()

        def gather(c, rows, sem):
            return pltpu.async_copy(h_hbm.at[cid].at[sidx.at[c]], rows, sem)

        def scale(c, rows, rs, wrow):
            lane = lax.iota(jnp.int32, LANES)
            hidx = [jnp.full((LANES,), hl, jnp.int32) + 2 * cid
                    for hl in range(2)]
            for g in range(groups):
                el = g * LANES + lane
                s_i = sidx[c, pl.ds(g * LANES, LANES)]
                d_i = didx[c, pl.ds(g * LANES, LANES)]
                for hl in range(2):
                    a_s = plsc.load_gather(alpha_v, [s_i * 4 + hl])
                    a_d = plsc.load_gather(alpha_v, [d_i * 4 + 2 + hl])
                    e = a_s + a_d
                    e = jnp.where(e > 0, e, NEG_SLOPE * e)
                    w = jnp.exp(e)
                    plsc.store_scatter(wrow, [el, hidx[hl]], w)
                for k in range(LANES):
                    ek = g * LANES + k
                    wv = wrow[ek, pl.ds(0, LANES)]
                    for hl in range(2):
                        wk = _lane_splat(wv, hidx[hl])
                        base = hl * OUT
                        rs[ek, pl.ds(base, LANES)] = (
                            rows[ek, pl.ds(base, LANES)] * wk)
                        rs[ek, pl.ds(base + LANES, LANES)] = (
                            rows[ek, pl.ds(base + LANES, LANES)] * wk)

        def sissue(c, rs, wrow, sem):
            pltpu.async_copy(rs, acch.at[didx.at[c]], sem, add=True)
            pltpu.async_copy(wrow, accd.at[didx.at[c]], sem, add=True)

        def swait(rs, wrow, sem):
            pltpu.make_async_copy(rs, acch.at[didx.at[0]], sem).wait()
            pltpu.make_async_copy(wrow, accd.at[didx.at[0]], sem).wait()

        def blk(b, _):
            pltpu.sync_copy(src_hbm.at[sid, b], sidx)
            pltpu.sync_copy(dst_hbm.at[sid, b], didx)
            gather(0, rows_a, sem_a).wait()

            def body(i, _):
                c = 2 * i
                db = gather(c + 1, rows_b, sem_b)

                @pl.when((b > 0) | (i > 0))
                def _():
                    swait(rs_a, wr_a, sca)

                scale(c, rows_a, rs_a, wr_a)
                sissue(c, rs_a, wr_a, sca)
                da = gather(jnp.minimum(c + 2, GBLK - 1), rows_a, sem_a)
                db.wait()

                @pl.when((b > 0) | (i > 0))
                def _():
                    swait(rs_b, wr_b, scb)

                scale(c + 1, rows_b, rs_b, wr_b)
                sissue(c + 1, rs_b, wr_b, scb)
                da.wait()
                return 0

            lax.fori_loop(0, GBLK // 2, body, 0)
            return 0

        lax.fori_loop(0, nblk, blk, 0)
        swait(rs_a, wr_a, sca)
        swait(rs_b, wr_b, scb)
        plsc.subcore_barrier()
        pltpu.sync_copy(acch.at[pl.ds(rbase, rows_tile)],
                        outh_hbm.at[cid, pl.ds(rbase, rows_tile)])
        pltpu.sync_copy(accd.at[pl.ds(rbase, rows_tile)],
                        outd_hbm.at[cid, pl.ds(rbase, rows_tile)])

    return k(h2, alpha2, src4d, dst4d, hinit2, dinit2)


def kernel(x, edge_index, batch, y, num_graphs, W_gat, a_src, a_dst, Wm, Wi,
           Wh, bi, bh, W1, b1, W2, b2):
    n = x.shape[0]
    e_cnt = edge_index.shape[1]
    g_cnt = y.shape[0]
    del num_graphs

    eye4 = jnp.eye(HEADS, dtype=jnp.float32)
    asel = jnp.einsum('hk,hg->hkg', _f32(a_src), eye4).reshape(HID, HEADS)
    adsel = jnp.einsum('hk,hg->hkg', _f32(a_dst), eye4).reshape(HID, HEADS)
    rsel = jnp.repeat(eye4, OUT, axis=1).reshape(HEADS, HID)
    psel = jnp.concatenate(
        [eye4, jnp.zeros((HEADS, LANES - HEADS), jnp.float32)], axis=1)
    psum = jnp.concatenate(
        [jnp.repeat(eye4, OUT, axis=1).reshape(HEADS, HID),
         jnp.zeros((LANES - HEADS, HID), jnp.float32)], axis=0)

    n_pad = ((n + 127) // 128) * 128
    chunks_tile = e_cnt // (NS * CHUNK)
    src = edge_index[0].astype(jnp.int32)
    dst = edge_index[1].astype(jnp.int32)
    mchunk = 100
    src3d = src.reshape(NS, e_cnt // (NS * mchunk), mchunk)
    dst3d = dst.reshape(NS, e_cnt // (NS * mchunk), mchunk)
    src4d = src.reshape(NS, NGBLK, GBLK, CHUNK)
    dst4d = dst.reshape(NS, NGBLK, GBLK, CHUNK)
    x_p = jnp.pad(_f32(x), ((0, n_pad - n), (0, 0)))
    batch2d = jnp.pad(batch.astype(jnp.int32), (0, n_pad - n),
                      constant_values=g_cnt).reshape(1, n_pad)
    y2d = _f32(y).reshape(g_cnt, 1)
    bi2 = _f32(bi).reshape(N_BLOCKS, 1, 3 * HID)
    bh2 = _f32(bh).reshape(N_BLOCKS, 1, 3 * HID)
    b12 = _f32(b1).reshape(1, 64)
    b22 = _f32(b2).reshape(1, 1)

    h2, alpha2, hinit2, dinit2 = _tc_call(
        _tc_a_body,
        (jax.ShapeDtypeStruct((NC, n_pad, HALF), jnp.float32),
         jax.ShapeDtypeStruct((NC, n_pad, HEADS), jnp.float32),
         jax.ShapeDtypeStruct((NC, n_pad, HALF), jnp.float32),
         jax.ShapeDtypeStruct((NC, n_pad, LANES), jnp.float32)),
        x_p, _f32(W_gat), asel, adsel, rsel, psel)

    acch, accd = _gat_edges_sc(h2, alpha2.reshape(NC, n_pad * HEADS), src4d,
                               dst4d, hinit2, dinit2, n_pad)

    h, v2 = _tc_call(
        _tc_b_body,
        (jax.ShapeDtypeStruct((n_pad, HID), jnp.float32),
         jax.ShapeDtypeStruct((NC, n_pad, HALF), jnp.float32)),
        acch, accd, _f32(Wm[0]), psum)
    hist = h

    for i in range(N_BLOCKS):
        accm = _segsum_sc(v2, src3d, dst3d, n_pad)
        if i + 1 < N_BLOCKS:
            h, hist, v2 = _tc_call(
                _tc_gru_body,
                (jax.ShapeDtypeStruct((n_pad, HID), jnp.float32),
                 jax.ShapeDtypeStruct((n_pad, HID), jnp.float32),
                 jax.ShapeDtypeStruct((NC, n_pad, HALF), jnp.float32)),
                accm, h, hist, _f32(Wi[i]), _f32(Wh[i]), bi2[i], bh2[i],
                _f32(Wm[i + 1]))
        else:
            scores2d, loss2d = _tc_call(
                _tc_gru_last_body,
                (jax.ShapeDtypeStruct((g_cnt, 1), jnp.float32),
                 jax.ShapeDtypeStruct((1, 1), jnp.float32)),
                accm, h, hist, _f32(Wi[i]), _f32(Wh[i]), bi2[i], bh2[i],
                batch2d, y2d, _f32(W1), b12, _f32(W2), b22)

    return scores2d.reshape(g_cnt), loss2d.reshape(())
